# initial kernel scaffold (unmeasured)
import jax
import jax.numpy as jnp
from jax import lax
from jax.experimental import pallas as pl
from jax.experimental.pallas import tpu as pltpu

N_DEV = 4
M_PER = 1024
K_DIM = 4096
N_PER = 2048
K_BLK = 512
K_STEPS = K_DIM // K_BLK


def kernel(x, w_mat):
    def body(x_ref, w_ref, out_ref, acc_ref, send_sems, recv_sems):
        p = pl.program_id(0)
        k = pl.program_id(1)
        my = lax.axis_index("i")

        @pl.when((p == 0) & (k == 0))
        def _():
            barrier = pltpu.get_barrier_semaphore()
            for d in range(1, N_DEV):
                pl.semaphore_signal(
                    barrier, inc=1,
                    device_id=((my + d) % N_DEV,),
                    device_id_type=pl.DeviceIdType.MESH,
                )
            pl.semaphore_wait(barrier, N_DEV - 1)

        prod = jnp.dot(x_ref[...], w_ref[...],
                       preferred_element_type=jnp.float32)

        @pl.when(k == 0)
        def _():
            acc_ref[p] = prod

        @pl.when(k > 0)
        def _():
            acc_ref[p] = acc_ref[p] + prod

        @pl.when(k == K_STEPS - 1)
        def _():
            acc_ref[p] = jnp.maximum(acc_ref[p], 0.0)

            @pl.when(p == my)
            def _():
                out_ref[pl.ds(my * M_PER, M_PER), :] = acc_ref[p]

            @pl.when(p != my)
            def _():
                rdma = pltpu.make_async_remote_copy(
                    src_ref=acc_ref.at[p],
                    dst_ref=out_ref.at[pl.ds(my * M_PER, M_PER), :],
                    send_sem=send_sems.at[p],
                    recv_sem=recv_sems.at[my],
                    device_id=(p,),
                    device_id_type=pl.DeviceIdType.MESH,
                )
                rdma.start()

        @pl.when((p == N_DEV - 1) & (k == K_STEPS - 1))
        def _():
            for d in range(1, N_DEV):
                src = (my + d) % N_DEV
                recv = pltpu.make_async_remote_copy(
                    src_ref=acc_ref.at[0],
                    dst_ref=out_ref.at[pl.ds(src * M_PER, M_PER), :],
                    send_sem=send_sems.at[0],
                    recv_sem=recv_sems.at[src],
                    device_id=(src,),
                    device_id_type=pl.DeviceIdType.MESH,
                )
                recv.wait_recv()
            for d in range(1, N_DEV):
                tgt = (my + d) % N_DEV
                send = pltpu.make_async_remote_copy(
                    src_ref=acc_ref.at[tgt],
                    dst_ref=out_ref.at[pl.ds(my * M_PER, M_PER), :],
                    send_sem=send_sems.at[tgt],
                    recv_sem=recv_sems.at[my],
                    device_id=(tgt,),
                    device_id_type=pl.DeviceIdType.MESH,
                )
                send.wait_send()

    return pl.pallas_call(
        body,
        grid=(N_DEV, K_STEPS),
        in_specs=[
            pl.BlockSpec((M_PER, K_BLK), lambda p, k: (0, k)),
            pl.BlockSpec((K_BLK, N_PER), lambda p, k: (k, p)),
        ],
        out_specs=pl.BlockSpec((N_DEV * M_PER, N_PER), lambda p, k: (0, 0)),
        out_shape=jax.ShapeDtypeStruct((N_DEV * M_PER, N_PER), jnp.float32),
        scratch_shapes=[
            pltpu.VMEM((N_DEV, M_PER, N_PER), jnp.float32),
            pltpu.SemaphoreType.DMA((N_DEV,)),
            pltpu.SemaphoreType.DMA((N_DEV,)),
        ],
        compiler_params=pltpu.CompilerParams(
            collective_id=0,
            dimension_semantics=("arbitrary", "arbitrary"),
        ),
    )(x, w_mat)


# baseline (device time: 468369 ns/iter reference)
import jax
import jax.numpy as jnp
from jax import lax
from jax.experimental import pallas as pl
from jax.experimental.pallas import tpu as pltpu

N_DEV = 4
M_PER = 1024
K_DIM = 4096
N_PER = 2048
K_BLK = 256
K_STEPS = K_DIM // K_BLK


def kernel(x, w_mat):
    def body(x_ref, w_ref, out_ref, acc_ref, send_sems, recv_sems):
        p = pl.program_id(0)
        k = pl.program_id(1)
        my = lax.axis_index("i")
        slot = p % 2

        def local_copy(piece):
            return pltpu.make_async_copy(
                acc_ref.at[piece % 2],
                out_ref.at[pl.ds(my * M_PER, M_PER), :],
                send_sems.at[piece],
            )

        def send_desc(piece):
            return pltpu.make_async_remote_copy(
                src_ref=acc_ref.at[piece % 2],
                dst_ref=out_ref.at[pl.ds(my * M_PER, M_PER), :],
                send_sem=send_sems.at[piece],
                recv_sem=recv_sems.at[my],
                device_id=(piece,),
                device_id_type=pl.DeviceIdType.MESH,
            )

        @pl.when((p == 0) & (k == 0))
        def _():
            barrier = pltpu.get_barrier_semaphore()
            for d in range(1, N_DEV):
                pl.semaphore_signal(
                    barrier, inc=1,
                    device_id=((my + d) % N_DEV,),
                    device_id_type=pl.DeviceIdType.MESH,
                )
            pl.semaphore_wait(barrier, N_DEV - 1)

        @pl.when((k == 0) & (p >= 2))
        def _():
            prev = p - 2

            @pl.when(prev == my)
            def _():
                local_copy(prev).wait()

            @pl.when(prev != my)
            def _():
                send_desc(prev).wait_send()

        prod = jnp.dot(x_ref[...], w_ref[...],
                       preferred_element_type=jnp.float32)

        @pl.when(k == 0)
        def _():
            acc_ref[slot] = prod

        @pl.when(k > 0)
        def _():
            acc_ref[slot] = acc_ref[slot] + prod

        @pl.when(k == K_STEPS - 1)
        def _():
            acc_ref[slot] = jnp.maximum(acc_ref[slot], 0.0)

            @pl.when(p == my)
            def _():
                local_copy(p).start()

            @pl.when(p != my)
            def _():
                send_desc(p).start()

        @pl.when((p == N_DEV - 1) & (k == K_STEPS - 1))
        def _():
            for d in range(1, N_DEV):
                src = (my + d) % N_DEV
                recv = pltpu.make_async_remote_copy(
                    src_ref=acc_ref.at[0],
                    dst_ref=out_ref.at[pl.ds(src * M_PER, M_PER), :],
                    send_sem=send_sems.at[0],
                    recv_sem=recv_sems.at[src],
                    device_id=(src,),
                    device_id_type=pl.DeviceIdType.MESH,
                )
                recv.wait_recv()
            for q in (N_DEV - 2, N_DEV - 1):
                @pl.when(q == my)
                def _(q=q):
                    local_copy(q).wait()

                @pl.when(q != my)
                def _(q=q):
                    send_desc(q).wait_send()

    return pl.pallas_call(
        body,
        grid=(N_DEV, K_STEPS),
        in_specs=[
            pl.BlockSpec((M_PER, K_BLK), lambda p, k: (0, k)),
            pl.BlockSpec((K_BLK, N_PER), lambda p, k: (k, p)),
        ],
        out_specs=pl.BlockSpec(memory_space=pl.ANY),
        out_shape=jax.ShapeDtypeStruct((N_DEV * M_PER, N_PER), jnp.float32),
        scratch_shapes=[
            pltpu.VMEM((2, M_PER, N_PER), jnp.float32),
            pltpu.SemaphoreType.DMA((N_DEV,)),
            pltpu.SemaphoreType.DMA((N_DEV,)),
        ],
        compiler_params=pltpu.CompilerParams(
            collective_id=0,
            dimension_semantics=("arbitrary", "arbitrary"),
        ),
    )(x, w_mat)


# device time: 290766 ns/iter; 1.6108x vs baseline; 1.6108x over previous
import jax
import jax.numpy as jnp
from jax import lax
from jax.experimental import pallas as pl
from jax.experimental.pallas import tpu as pltpu

N_DEV = 4
M_PER = 1024
K_DIM = 4096
N_PER = 2048
K_BLK = 256
K_STEPS = K_DIM // K_BLK


def kernel(x, w_mat):
    my = lax.axis_index("i")
    perm = (my + 1 + jnp.arange(N_DEV, dtype=jnp.int32)) % N_DEV

    def body(perm_ref, x_ref, w_ref, out_ref, acc_ref, send_sems, recv_sems):
        p = pl.program_id(0)
        k = pl.program_id(1)
        my = lax.axis_index("i")
        slot = p % 2

        def send_desc(piece, tgt):
            return pltpu.make_async_remote_copy(
                src_ref=acc_ref.at[piece % 2],
                dst_ref=out_ref.at[pl.ds(my * M_PER, M_PER), :],
                send_sem=send_sems.at[piece],
                recv_sem=recv_sems.at[my],
                device_id=(tgt,),
                device_id_type=pl.DeviceIdType.MESH,
            )

        def self_copy():
            return pltpu.make_async_copy(
                acc_ref.at[(N_DEV - 1) % 2],
                out_ref.at[pl.ds(my * M_PER, M_PER), :],
                send_sems.at[N_DEV - 1],
            )

        @pl.when((p == 0) & (k == 0))
        def _():
            barrier = pltpu.get_barrier_semaphore()
            for d in range(1, N_DEV):
                pl.semaphore_signal(
                    barrier, inc=1,
                    device_id=((my + d) % N_DEV,),
                    device_id_type=pl.DeviceIdType.MESH,
                )
            pl.semaphore_wait(barrier, N_DEV - 1)

        @pl.when((k == 0) & (p >= 2))
        def _():
            send_desc(p - 2, 0).wait_send()

        prod = jnp.dot(x_ref[...], w_ref[...],
                       preferred_element_type=jnp.float32)

        @pl.when(k == 0)
        def _():
            acc_ref[slot] = prod

        @pl.when(k > 0)
        def _():
            acc_ref[slot] = acc_ref[slot] + prod

        @pl.when(k == K_STEPS - 1)
        def _():
            acc_ref[slot] = jnp.maximum(acc_ref[slot], 0.0)

            @pl.when(p < N_DEV - 1)
            def _():
                send_desc(p, perm_ref[p]).start()

            @pl.when(p == N_DEV - 1)
            def _():
                self_copy().start()

        @pl.when((p == N_DEV - 1) & (k == K_STEPS - 1))
        def _():
            for d in range(1, N_DEV):
                src = (my + d) % N_DEV
                recv = pltpu.make_async_remote_copy(
                    src_ref=acc_ref.at[0],
                    dst_ref=out_ref.at[pl.ds(src * M_PER, M_PER), :],
                    send_sem=send_sems.at[0],
                    recv_sem=recv_sems.at[src],
                    device_id=(src,),
                    device_id_type=pl.DeviceIdType.MESH,
                )
                recv.wait_recv()
            send_desc(N_DEV - 2, 0).wait_send()
            self_copy().wait()

    grid_spec = pltpu.PrefetchScalarGridSpec(
        num_scalar_prefetch=1,
        grid=(N_DEV, K_STEPS),
        in_specs=[
            pl.BlockSpec((M_PER, K_BLK), lambda p, k, perm: (0, k)),
            pl.BlockSpec((K_BLK, N_PER), lambda p, k, perm: (k, perm[p])),
        ],
        out_specs=pl.BlockSpec(memory_space=pl.ANY),
        scratch_shapes=[
            pltpu.VMEM((2, M_PER, N_PER), jnp.float32),
            pltpu.SemaphoreType.DMA((N_DEV,)),
            pltpu.SemaphoreType.DMA((N_DEV,)),
        ],
    )

    return pl.pallas_call(
        body,
        grid_spec=grid_spec,
        out_shape=jax.ShapeDtypeStruct((N_DEV * M_PER, N_PER), jnp.float32),
        compiler_params=pltpu.CompilerParams(
            collective_id=0,
            dimension_semantics=("arbitrary", "arbitrary"),
        ),
    )(perm, x, w_mat)
